# baseline (speedup 1.0000x reference)
import functools

import numpy as np
import jax
import jax.numpy as jnp
from jax import lax
from jax.experimental import pallas as pl
from jax.experimental.pallas import tpu as pltpu
from jax.experimental.pallas import tpu_sc as plsc

_RB = 512


_NBUF = 3


def _tc_stats_body(x_hbm, t_ref, lse_ref, rs_ref, xt_ref, buf, sems, *, nblk):
    i = pl.program_id(0)

    @pl.when(i == 0)
    def _():
        for b in range(_NBUF - 1):
            pltpu.make_async_copy(
                x_hbm.at[pl.ds(b * _RB, _RB), :], buf.at[b], sems.at[b]
            ).start()

    nxt = i + _NBUF - 1
    for b in range(_NBUF):
        @pl.when(jnp.logical_and(nxt < nblk, lax.rem(nxt, _NBUF) == b))
        def _(b=b):
            pltpu.make_async_copy(
                x_hbm.at[pl.ds(nxt * _RB, _RB), :], buf.at[b], sems.at[b]
            ).start()

    for b in range(_NBUF):
        @pl.when(lax.rem(i, _NBUF) == b)
        def _(b=b):
            pltpu.make_async_copy(
                x_hbm.at[pl.ds(i * _RB, _RB), :], buf.at[b], sems.at[b]
            ).wait()
            x = buf[b]
            rb, v = x.shape
            m = jnp.max(x, axis=1)
            se = jnp.sum(jnp.exp(x - m[:, None]), axis=1)
            t = t_ref[0, 0, :]
            cols = lax.broadcasted_iota(jnp.int32, (rb, v), 1)
            xt = jnp.sum(jnp.where(cols == t[:, None], x, 0.0), axis=1)
            lse_ref[0, 0, :] = m + jnp.log(se)
            rs_ref[0, 0, :] = jnp.sum(x, axis=1)
            xt_ref[0, 0, :] = xt


def _tc_stats(x, t3):
    n, v = x.shape
    nblk = n // _RB
    vec = jax.ShapeDtypeStruct((nblk, 1, _RB), jnp.float32)
    return pl.pallas_call(
        functools.partial(_tc_stats_body, nblk=nblk),
        grid=(nblk,),
        in_specs=[
            pl.BlockSpec(memory_space=pl.ANY),
            pl.BlockSpec((1, 1, _RB), lambda i: (i, 0, 0)),
        ],
        out_specs=[
            pl.BlockSpec((1, 1, _RB), lambda i: (i, 0, 0)),
            pl.BlockSpec((1, 1, _RB), lambda i: (i, 0, 0)),
            pl.BlockSpec((1, 1, _RB), lambda i: (i, 0, 0)),
        ],
        out_shape=[vec, vec, vec],
        scratch_shapes=[
            pltpu.VMEM((_NBUF, _RB, v), jnp.float32),
            pltpu.SemaphoreType.DMA((_NBUF,)),
        ],
    )(x, t3)


def _tc_combine_body(lse_ref, rs_ref, xt_ref, ns_ref, out_ref, *, c_smooth, v):
    lse = lse_ref[...]
    rs = rs_ref[...]
    xt = xt_ref[...]
    ns = ns_ref[...]
    w = ns * (c_smooth / (v - 1))
    src = 1.0 - v * w
    logp_t = xt - lse
    s_row = rs - v * lse
    ent = (v - 1.0) * (w * jnp.log(jnp.where(w > 0, w, 1.0))) \
        + src * jnp.log(jnp.where(src > 0, src, 1.0))
    cross = w * (s_row - logp_t) + src * logp_t
    out_ref[0, 0] = jnp.sum(ent - cross)


def _tc_combine(lse, rs, xt, ns3, c_smooth, v):
    return pl.pallas_call(
        functools.partial(_tc_combine_body, c_smooth=c_smooth, v=v),
        out_specs=pl.BlockSpec(memory_space=pltpu.SMEM),
        out_shape=jax.ShapeDtypeStruct((1, 1), jnp.float32),
    )(lse, rs, xt, ns3)


def _sc_fetch(table2d, row_idx, col_idx):
    info = plsc.get_sparse_core_info()
    nc, ns_sub = info.num_cores, info.num_subcores
    nw = nc * ns_sub
    n = row_idx.shape[0]
    bpw = n // nw
    mesh = plsc.VectorSubcoreMesh(core_axis_name="c", subcore_axis_name="s")

    nrow, ncol = table2d.shape
    view = (
        table2d.reshape(nrow // 8, 8, ncol // 128, 128)
        .swapaxes(1, 2)
        .reshape(nrow * ncol // 128, 128)
    )
    vrow = ((row_idx >> 3) * (ncol // 128) + (col_idx >> 7)) * 8 + (row_idx & 7)
    vlane = col_idx & 127

    @functools.partial(
        pl.kernel,
        mesh=mesh,
        out_type=jax.ShapeDtypeStruct((n,), jnp.float32),
        scratch_types=[
            pltpu.VMEM((bpw,), jnp.int32),
            pltpu.VMEM((bpw,), jnp.int32),
            pltpu.VMEM((bpw, 128), jnp.float32),
            pltpu.VMEM((bpw,), jnp.float32),
            pltpu.SemaphoreType.DMA,
        ],
    )
    def k(view_hbm, vr_hbm, ln_hbm, out_hbm, vr_v, ln_v, rows_v, vals_v, sem):
        wid = lax.axis_index("s") * nc + lax.axis_index("c")
        base = wid * bpw
        pltpu.sync_copy(vr_hbm.at[pl.ds(base, bpw)], vr_v)
        pltpu.sync_copy(ln_hbm.at[pl.ds(base, bpw)], ln_v)
        pltpu.async_copy(view_hbm.at[vr_v], rows_v, sem).wait()
        lane = lax.iota(jnp.int32, 16)
        for g in range(bpw // 16):
            o = g * 16
            lnv = ln_v[pl.ds(o, 16)]
            acc = jnp.zeros((16,), jnp.float32)
            for i in range(16):
                ln = lnv[i]
                w = rows_v[o + i, pl.ds((ln >> 4) * 16, 16)]
                v16 = lax.gather(
                    w,
                    jnp.full((16, 1), ln & 15, jnp.int32),
                    lax.GatherDimensionNumbers(
                        offset_dims=(), collapsed_slice_dims=(0,),
                        start_index_map=(0,)),
                    (1,),
                    mode=lax.GatherScatterMode.PROMISE_IN_BOUNDS,
                )
                acc = jnp.where(lane == i, v16, acc)
            vals_v[pl.ds(o, 16)] = acc
        pltpu.sync_copy(vals_v, out_hbm.at[pl.ds(base, bpw)])

    return k(view, vrow, vlane)


def kernel(inputs, targets, placeholder, labels, matric):
    nb, nt = targets.shape
    n = nb * nt
    v = inputs.shape[-1]
    x = inputs.reshape(n, v)
    t = targets.reshape(-1)

    seg_count = labels.shape[0]
    seg_len = labels.shape[1] + 1
    c_smooth = float(1.0 - np.power(1.0 - 0.1, 1.0 / np.float64(seg_len)))

    prev = jnp.concatenate(
        [jnp.full((nb, 1), n - 1, dtype=targets.dtype), targets[:, : nt - 1]],
        axis=1,
    ).reshape(-1)
    nblk = n // _RB
    t3 = t.reshape(nblk, 1, _RB)
    lse, rs, xt = _tc_stats(x, t3)
    need_smoothed = _sc_fetch(matric, prev, t)
    ns3 = need_smoothed.reshape(nblk, 1, _RB)
    total = _tc_combine(lse, rs, xt, ns3, c_smooth, v)[0, 0]
    return total / jnp.float32(seg_count * seg_len)


# speedup vs baseline: 1.0184x; 1.0184x over previous
import functools

import numpy as np
import jax
import jax.numpy as jnp
from jax import lax
from jax.experimental import pallas as pl
from jax.experimental.pallas import tpu as pltpu
from jax.experimental.pallas import tpu_sc as plsc

_RB = 512


def _tc_stats_body(x_ref, t_ref, lse_ref, rs_ref, xt_ref):
    x = x_ref[...]
    rb, v = x.shape
    m = jnp.max(x, axis=1)
    se = jnp.sum(jnp.exp(x - m[:, None]), axis=1)
    t = t_ref[0, 0, :]
    cols = lax.broadcasted_iota(jnp.int32, (rb, v), 1)
    xt = jnp.sum(jnp.where(cols == t[:, None], x, 0.0), axis=1)
    lse_ref[0, 0, :] = m + jnp.log(se)
    rs_ref[0, 0, :] = jnp.sum(x, axis=1)
    xt_ref[0, 0, :] = xt


def _tc_stats(x, t3):
    n, v = x.shape
    nblk = n // _RB
    vec = jax.ShapeDtypeStruct((nblk, 1, _RB), jnp.float32)
    return pl.pallas_call(
        _tc_stats_body,
        grid=(nblk,),
        in_specs=[
            pl.BlockSpec((_RB, v), lambda i: (i, 0)),
            pl.BlockSpec((1, 1, _RB), lambda i: (i, 0, 0)),
        ],
        out_specs=[
            pl.BlockSpec((1, 1, _RB), lambda i: (i, 0, 0)),
            pl.BlockSpec((1, 1, _RB), lambda i: (i, 0, 0)),
            pl.BlockSpec((1, 1, _RB), lambda i: (i, 0, 0)),
        ],
        out_shape=[vec, vec, vec],
    )(x, t3)


def _tc_combine_body(lse_ref, rs_ref, xt_ref, ns_ref, out_ref, *, c_smooth, v):
    lse = lse_ref[...]
    rs = rs_ref[...]
    xt = xt_ref[...]
    ns = ns_ref[...]
    w = ns * (c_smooth / (v - 1))
    src = 1.0 - v * w
    logp_t = xt - lse
    s_row = rs - v * lse
    ent = (v - 1.0) * (w * jnp.log(jnp.where(w > 0, w, 1.0))) \
        + src * jnp.log(jnp.where(src > 0, src, 1.0))
    cross = w * (s_row - logp_t) + src * logp_t
    out_ref[0, 0] = jnp.sum(ent - cross)


def _tc_combine(lse, rs, xt, ns3, c_smooth, v):
    return pl.pallas_call(
        functools.partial(_tc_combine_body, c_smooth=c_smooth, v=v),
        out_specs=pl.BlockSpec(memory_space=pltpu.SMEM),
        out_shape=jax.ShapeDtypeStruct((1, 1), jnp.float32),
    )(lse, rs, xt, ns3)


def _sc_fetch(table2d, row_idx, col_idx):
    info = plsc.get_sparse_core_info()
    nc, ns_sub = info.num_cores, info.num_subcores
    nw = nc * ns_sub
    n = row_idx.shape[0]
    bpw = n // nw
    mesh = plsc.VectorSubcoreMesh(core_axis_name="c", subcore_axis_name="s")

    nrow, ncol = table2d.shape
    view = (
        table2d.reshape(nrow // 8, 8, ncol // 128, 128)
        .swapaxes(1, 2)
        .reshape(nrow * ncol // 128, 128)
    )
    vrow = ((row_idx >> 3) * (ncol // 128) + (col_idx >> 7)) * 8 + (row_idx & 7)
    vlane = col_idx & 127

    @functools.partial(
        pl.kernel,
        mesh=mesh,
        out_type=jax.ShapeDtypeStruct((n,), jnp.float32),
        scratch_types=[
            pltpu.VMEM((bpw,), jnp.int32),
            pltpu.VMEM((bpw,), jnp.int32),
            pltpu.VMEM((bpw, 128), jnp.float32),
            pltpu.VMEM((bpw,), jnp.float32),
            pltpu.SemaphoreType.DMA,
        ],
    )
    def k(view_hbm, vr_hbm, ln_hbm, out_hbm, vr_v, ln_v, rows_v, vals_v, sem):
        wid = lax.axis_index("s") * nc + lax.axis_index("c")
        base = wid * bpw
        pltpu.sync_copy(vr_hbm.at[pl.ds(base, bpw)], vr_v)
        pltpu.sync_copy(ln_hbm.at[pl.ds(base, bpw)], ln_v)
        pltpu.async_copy(view_hbm.at[vr_v], rows_v, sem).wait()
        lane = lax.iota(jnp.int32, 16)
        for g in range(bpw // 16):
            o = g * 16
            lnv = ln_v[pl.ds(o, 16)]
            acc = jnp.zeros((16,), jnp.float32)
            for i in range(16):
                ln = lnv[i]
                w = rows_v[o + i, pl.ds((ln >> 4) * 16, 16)]
                v16 = lax.gather(
                    w,
                    jnp.full((16, 1), ln & 15, jnp.int32),
                    lax.GatherDimensionNumbers(
                        offset_dims=(), collapsed_slice_dims=(0,),
                        start_index_map=(0,)),
                    (1,),
                    mode=lax.GatherScatterMode.PROMISE_IN_BOUNDS,
                )
                acc = jnp.where(lane == i, v16, acc)
            vals_v[pl.ds(o, 16)] = acc
        pltpu.sync_copy(vals_v, out_hbm.at[pl.ds(base, bpw)])

    return k(view, vrow, vlane)


def kernel(inputs, targets, placeholder, labels, matric):
    nb, nt = targets.shape
    n = nb * nt
    v = inputs.shape[-1]
    x = inputs.reshape(n, v)
    t = targets.reshape(-1)

    seg_count = labels.shape[0]
    seg_len = labels.shape[1] + 1
    c_smooth = float(1.0 - np.power(1.0 - 0.1, 1.0 / np.float64(seg_len)))

    prev = jnp.concatenate(
        [jnp.full((nb, 1), n - 1, dtype=targets.dtype), targets[:, : nt - 1]],
        axis=1,
    ).reshape(-1)
    nblk = n // _RB
    t3 = t.reshape(nblk, 1, _RB)
    lse, rs, xt = _tc_stats(x, t3)
    need_smoothed = _sc_fetch(matric, prev, t)
    ns3 = need_smoothed.reshape(nblk, 1, _RB)
    total = _tc_combine(lse, rs, xt, ns3, c_smooth, v)[0, 0]
    return total / jnp.float32(seg_count * seg_len)
